# baseline (speedup 1.0000x reference)
import jax
import jax.numpy as jnp
from jax.experimental import pallas as pl
from jax.experimental.pallas import tpu as pltpu

B, F_IN, F_OUT = 64, 4096, 4096
KB = 1024
OB = 1024


def _mm_body(x_ref, w_ref, m_ref, o_ref):
    k = pl.program_id(1)
    wm = w_ref[...] * m_ref[...].astype(jnp.float32)
    xb = x_ref[:, pl.ds(k * KB, KB)]
    acc = jax.lax.dot_general(
        xb, wm, (((1,), (1,)), ((), ())),
        preferred_element_type=jnp.float32)

    @pl.when(k == 0)
    def _init():
        o_ref[...] = acc

    @pl.when(k != 0)
    def _acc():
        o_ref[...] += acc


def kernel(x, weight, mask):
    grid = (F_OUT // OB, F_IN // KB)
    return pl.pallas_call(
        _mm_body,
        grid=grid,
        in_specs=[
            pl.BlockSpec((B, F_IN), lambda o, k: (0, 0)),
            pl.BlockSpec((OB, KB), lambda o, k: (o, k)),
            pl.BlockSpec((OB, KB), lambda o, k: (o, k)),
        ],
        out_specs=pl.BlockSpec((B, OB), lambda o, k: (0, o)),
        out_shape=jax.ShapeDtypeStruct((B, F_OUT), jnp.float32),
        compiler_params=pltpu.CompilerParams(
            dimension_semantics=("parallel", "arbitrary")),
    )(x, weight, mask)
